# baseline (speedup 1.0000x reference)
import jax
import jax.numpy as jnp
from jax import lax
from jax.experimental import pallas as pl
from jax.experimental.pallas import tpu as pltpu
from jax.experimental.pallas import tpu_sc as plsc

N = 10000
NP = 10240
E = 320000
D = 128
H = D // 2
NC = 2
NS = 16
NW = NC * NS
CHUNK = 80
NBUF = 8
NCH_A = -(-(-(-(E // NS) // CHUNK)) // NBUF) * NBUF
CH_D = 80
NCH_D = E // NW // CH_D
RPT = NP // NS
ZROWS = 128
CW = 16


def _sc_aggregate(hlo, hhi, src3, dst3):
    mesh = plsc.VectorSubcoreMesh(core_axis_name="c", subcore_axis_name="s")
    out_type = (jax.ShapeDtypeStruct((NP, H), jnp.float32),
                jax.ShapeDtypeStruct((NP, H), jnp.float32))
    scratch = [
        pltpu.VMEM((NCH_A, CHUNK), jnp.int32),
        pltpu.VMEM((NCH_A, CHUNK), jnp.int32),
        [pltpu.VMEM((CHUNK, H), jnp.float32) for _ in range(NBUF)],
        pltpu.VMEM((ZROWS, H), jnp.float32),
        pltpu.VMEM_SHARED((NP, H), jnp.float32),
        [pltpu.SemaphoreType.DMA for _ in range(NBUF)],
    ]

    def body(lo_hbm, hi_hbm, src_hbm, dst_hbm, olo_hbm, ohi_hbm,
             src_v, dst_v, rows, zrow_v, acc_sh, gsem):
        cid = lax.axis_index("c")
        sid = lax.axis_index("s")

        zeros16 = jnp.zeros((16,), jnp.float32)

        @pl.loop(0, ZROWS)
        def _(r):
            for c in range(H // 16):
                zrow_v[r, pl.ds(c * 16, 16)] = zeros16

        @pl.loop(0, RPT // ZROWS)
        def _(r):
            pltpu.sync_copy(zrow_v,
                            acc_sh.at[pl.ds(sid * RPT + r * ZROWS, ZROWS)])

        pltpu.sync_copy(src_hbm.at[sid], src_v)
        pltpu.sync_copy(dst_hbm.at[sid], dst_v)

        plsc.subcore_barrier()

        def run(table):
            def gather(j, b):
                pltpu.make_async_copy(table.at[src_v.at[j]], rows[b],
                                      gsem[b]).start()

            def gather_wait(j, b):
                pltpu.make_async_copy(table.at[src_v.at[j]], rows[b],
                                      gsem[b]).wait()

            for b in range(NBUF):
                gather(b, b)

            @pl.loop(0, NCH_A, step=NBUF)
            def _(j):
                for b in range(NBUF):
                    gather_wait(j + b, b)
                    pltpu.sync_copy(rows[b], acc_sh.at[dst_v.at[j + b]],
                                    add=True)

                    @pl.when(j + b + NBUF < NCH_A)
                    def _(b=b):
                        gather(j + b + NBUF, b)

        @pl.when(cid == 0)
        def _():
            run(lo_hbm)

        @pl.when(cid == 1)
        def _():
            run(hi_hbm)

        plsc.subcore_barrier()

        @pl.when(cid == 0)
        def _():
            pltpu.sync_copy(acc_sh.at[pl.ds(sid * RPT, RPT)],
                            olo_hbm.at[pl.ds(sid * RPT, RPT)])

        @pl.when(cid == 1)
        def _():
            pltpu.sync_copy(acc_sh.at[pl.ds(sid * RPT, RPT)],
                            ohi_hbm.at[pl.ds(sid * RPT, RPT)])

    k = pl.kernel(body, out_type=out_type, mesh=mesh, scratch_types=scratch,
                  compiler_params=pltpu.CompilerParams(
                      use_tc_tiling_on_sc=False))
    return k(hlo, hhi, src3, dst3)


def _sc_degree(dst3):
    mesh = plsc.VectorSubcoreMesh(core_axis_name="c", subcore_axis_name="s")
    scratch = [
        pltpu.VMEM((NCH_D, CH_D), jnp.int32),
        pltpu.VMEM((CH_D, CW), jnp.float32),
        pltpu.VMEM((ZROWS, CW), jnp.float32),
        pltpu.VMEM_SHARED((NP, CW), jnp.float32),
    ]

    def body(dst_hbm, cnt_hbm, dst_v, ones_v, zcnt_v, cnt_sh):
        cid = lax.axis_index("c")
        sid = lax.axis_index("s")
        wid = cid * NS + sid

        zeros16 = jnp.zeros((16,), jnp.float32)
        ones16 = jnp.ones((16,), jnp.float32)

        @pl.loop(0, ZROWS)
        def _(r):
            zcnt_v[r, pl.ds(0, 16)] = zeros16

        @pl.loop(0, CH_D)
        def _(r):
            ones_v[r, pl.ds(0, 16)] = ones16

        @pl.loop(0, RPT // ZROWS)
        def _(r):
            pltpu.sync_copy(zcnt_v,
                            cnt_sh.at[pl.ds(sid * RPT + r * ZROWS, ZROWS)])

        pltpu.sync_copy(dst_hbm.at[wid], dst_v)

        plsc.subcore_barrier()

        @pl.loop(0, NCH_D)
        def _(j):
            pltpu.sync_copy(ones_v, cnt_sh.at[dst_v.at[j]], add=True)

        plsc.subcore_barrier()

        pltpu.sync_copy(cnt_sh.at[pl.ds(sid * RPT, RPT)],
                        cnt_hbm.at[pl.ds(cid * NP + sid * RPT, RPT)])

    k = pl.kernel(body, out_type=jax.ShapeDtypeStruct((NC * NP, CW),
                                                      jnp.float32),
                  mesh=mesh, scratch_types=scratch,
                  compiler_params=pltpu.CompilerParams(
                      use_tc_tiling_on_sc=False))
    return k(dst3)


_B = 1024


def _sage_block(plo, phi, c0, c1, hlo, hhi, wl_lo, wl_hi, bl, wr_lo, wr_hi):
    cnt = jnp.maximum(c0[0][:, :1] + c1[0][:, :1], 1.0)
    inv = 1.0 / cnt
    out = (jnp.dot(plo[...] * inv, wl_lo[...],
                   preferred_element_type=jnp.float32)
           + jnp.dot(phi[...] * inv, wl_hi[...],
                     preferred_element_type=jnp.float32)
           + jnp.dot(hlo[...], wr_lo[...],
                     preferred_element_type=jnp.float32)
           + jnp.dot(hhi[...], wr_hi[...],
                     preferred_element_type=jnp.float32)
           + bl[...])
    return jnp.maximum(out, 0.0)


def _combine_body(plo, phi, c0, c1, hlo, hhi, wl_lo, wl_hi, bl, wr_lo,
                  wr_hi, olo_ref, ohi_ref):
    res = _sage_block(plo, phi, c0, c1, hlo, hhi, wl_lo, wl_hi, bl,
                      wr_lo, wr_hi)
    olo_ref[...] = res[:, :H]
    ohi_ref[...] = res[:, H:]


def _head_body(plo, phi, c0, c1, hlo, hhi, wl_lo, wl_hi, bl, wr_lo, wr_hi,
               wh1, bh1, wh2, bh2, o_ref):
    res = _sage_block(plo, phi, c0, c1, hlo, hhi, wl_lo, wl_hi, bl,
                      wr_lo, wr_hi)
    t = jnp.maximum(
        jnp.dot(res, wh1[...], preferred_element_type=jnp.float32)
        + bh1[...], 0.0)
    o_ref[...] = (jnp.dot(t, wh2[...], preferred_element_type=jnp.float32)
                  + bh2[...])


def _tc_combine(plo, phi, cnt, hlo, hhi, WlT, bl, WrT, head=None):
    c3 = cnt.reshape(NC, NP, CW)
    bl2 = bl.reshape(1, D)
    full = lambda s: pl.BlockSpec(s, lambda i: (0,) * len(s))
    row = lambda w: pl.BlockSpec((_B, w), lambda i: (i, 0))
    in_specs = [
        row(H), row(H),
        pl.BlockSpec((1, _B, CW), lambda i: (0, i, 0)),
        pl.BlockSpec((1, _B, CW), lambda i: (1, i, 0)),
        row(H), row(H),
        full((H, D)), full((H, D)),
        full((1, D)),
        full((H, D)), full((H, D)),
    ]
    args = [plo, phi, c3, c3, hlo, hhi,
            WlT[:H], WlT[H:], bl2, WrT[:H], WrT[H:]]
    if head is None:
        fn = _combine_body
        out_specs = (row(H), row(H))
        out_shape = (jax.ShapeDtypeStruct((NP, H), jnp.float32),
                     jax.ShapeDtypeStruct((NP, H), jnp.float32))
    else:
        fn = _head_body
        wh1, bh1, wh2, bh2 = head
        in_specs += [full((D, D)), full((1, D)), full((D, D)), full((1, D))]
        args += [wh1, bh1, wh2, bh2]
        out_specs = row(D)
        out_shape = jax.ShapeDtypeStruct((NP, D), jnp.float32)
    return pl.pallas_call(
        fn,
        grid=(NP // _B,),
        in_specs=in_specs,
        out_specs=out_specs,
        out_shape=out_shape,
    )(*args)


def kernel(x, edge_index, Wl0, bl0, Wr0, Wl1, bl1, Wr1, Wl2, bl2, Wr2,
           Wh1, bh1, Wh2, bh2):
    npad = NCH_A * CHUNK - E // NS
    s2 = edge_index[0].reshape(NS, -1)
    d2 = edge_index[1].reshape(NS, -1)
    src16 = jnp.concatenate(
        [s2, jnp.zeros((NS, npad), jnp.int32)], axis=1
    ).reshape(NS, NCH_A, CHUNK)
    dst16 = jnp.concatenate(
        [d2, jnp.full((NS, npad), N, jnp.int32)], axis=1
    ).reshape(NS, NCH_A, CHUNK)
    dst32 = edge_index[1].reshape(NW, NCH_D, CH_D)
    xp = jnp.pad(x, ((0, NP - N), (0, 0)))
    xlo = xp[:, :H]
    xhi = xp[:, H:]

    Wh1T = Wh1.T
    Wh1Tp = jnp.pad(Wh1T, ((0, 0), (0, D - Wh1T.shape[1])))
    bh1p = jnp.pad(bh1, (0, D - bh1.shape[0])).reshape(1, D)
    Wh2T = Wh2.T
    Wh2Tp = jnp.pad(Wh2T, ((0, D - Wh2T.shape[0]), (0, D - Wh2T.shape[1])))
    bh2p = jnp.pad(bh2, (0, D - bh2.shape[0])).reshape(1, D)

    cnt = _sc_degree(dst32)
    plo, phi = _sc_aggregate(xlo, xhi, src16, dst16)
    hlo, hhi = _tc_combine(plo, phi, cnt, xlo, xhi, Wl0.T, bl0, Wr0.T)
    plo, phi = _sc_aggregate(hlo, hhi, src16, dst16)
    hlo, hhi = _tc_combine(plo, phi, cnt, hlo, hhi, Wl1.T, bl1, Wr1.T)
    plo, phi = _sc_aggregate(hlo, hhi, src16, dst16)
    y = _tc_combine(plo, phi, cnt, hlo, hhi, Wl2.T, bl2, Wr2.T,
                    head=(Wh1Tp, bh1p, Wh2Tp, bh2p))
    return y[:N, :4]


# speedup vs baseline: 1.4923x; 1.4923x over previous
import jax
import jax.numpy as jnp
from jax import lax
from jax.experimental import pallas as pl
from jax.experimental.pallas import tpu as pltpu
from jax.experimental.pallas import tpu_sc as plsc

N = 10000
NP = 10240
E = 320000
D = 128
H = D // 2
NC = 2
NS = 16
NW = NC * NS
CHUNK = 80
NBUF = 4
NCH_A = -(-(-(-(E // NS) // CHUNK)) // NBUF) * NBUF
CH_D = 80
NCH_D = E // NW // CH_D
RPT = NP // NS
ZROWS = 128
CW = 16


def _sc_aggregate(hlo, hhi, src3, dst3):
    mesh = plsc.VectorSubcoreMesh(core_axis_name="c", subcore_axis_name="s")
    out_type = (jax.ShapeDtypeStruct((NP, H), jnp.float32),
                jax.ShapeDtypeStruct((NP, H), jnp.float32))
    scratch = [
        pltpu.VMEM((NCH_A, CHUNK), jnp.int32),
        pltpu.VMEM((NCH_A, CHUNK), jnp.int32),
        [pltpu.VMEM((CHUNK, H), jnp.float32) for _ in range(NBUF)],
        pltpu.VMEM((ZROWS, H), jnp.float32),
        pltpu.VMEM_SHARED((NP, H), jnp.float32),
        [pltpu.SemaphoreType.DMA for _ in range(NBUF)],
    ]

    def body(lo_hbm, hi_hbm, src_hbm, dst_hbm, olo_hbm, ohi_hbm,
             src_v, dst_v, rows, zrow_v, acc_sh, gsem):
        cid = lax.axis_index("c")
        sid = lax.axis_index("s")

        zeros16 = jnp.zeros((16,), jnp.float32)

        @pl.loop(0, ZROWS)
        def _(r):
            for c in range(H // 16):
                zrow_v[r, pl.ds(c * 16, 16)] = zeros16

        @pl.loop(0, RPT // ZROWS)
        def _(r):
            pltpu.sync_copy(zrow_v,
                            acc_sh.at[pl.ds(sid * RPT + r * ZROWS, ZROWS)])

        pltpu.sync_copy(src_hbm.at[sid], src_v)
        pltpu.sync_copy(dst_hbm.at[sid], dst_v)

        plsc.subcore_barrier()

        def run(table):
            def gather(j, b):
                pltpu.make_async_copy(table.at[src_v.at[j]], rows[b],
                                      gsem[b]).start()

            def gather_wait(j, b):
                pltpu.make_async_copy(table.at[src_v.at[j]], rows[b],
                                      gsem[b]).wait()

            for b in range(NBUF):
                gather(b, b)

            @pl.loop(0, NCH_A, step=NBUF)
            def _(j):
                for b in range(NBUF):
                    gather_wait(j + b, b)
                    pltpu.sync_copy(rows[b], acc_sh.at[dst_v.at[j + b]],
                                    add=True)

                    @pl.when(j + b + NBUF < NCH_A)
                    def _(b=b):
                        gather(j + b + NBUF, b)

        @pl.when(cid == 0)
        def _():
            run(lo_hbm)

        @pl.when(cid == 1)
        def _():
            run(hi_hbm)

        plsc.subcore_barrier()

        @pl.when(cid == 0)
        def _():
            pltpu.sync_copy(acc_sh.at[pl.ds(sid * RPT, RPT)],
                            olo_hbm.at[pl.ds(sid * RPT, RPT)])

        @pl.when(cid == 1)
        def _():
            pltpu.sync_copy(acc_sh.at[pl.ds(sid * RPT, RPT)],
                            ohi_hbm.at[pl.ds(sid * RPT, RPT)])

    k = pl.kernel(body, out_type=out_type, mesh=mesh, scratch_types=scratch,
                  compiler_params=pltpu.CompilerParams(
                      use_tc_tiling_on_sc=False))
    return k(hlo, hhi, src3, dst3)


def _sc_degree(dst3):
    mesh = plsc.VectorSubcoreMesh(core_axis_name="c", subcore_axis_name="s")
    scratch = [
        pltpu.VMEM((NCH_D, CH_D), jnp.int32),
        pltpu.VMEM((CH_D, CW), jnp.float32),
        pltpu.VMEM((ZROWS, CW), jnp.float32),
        pltpu.VMEM_SHARED((NP, CW), jnp.float32),
    ]

    def body(dst_hbm, cnt_hbm, dst_v, ones_v, zcnt_v, cnt_sh):
        cid = lax.axis_index("c")
        sid = lax.axis_index("s")
        wid = cid * NS + sid

        zeros16 = jnp.zeros((16,), jnp.float32)
        ones16 = jnp.ones((16,), jnp.float32)

        @pl.loop(0, ZROWS)
        def _(r):
            zcnt_v[r, pl.ds(0, 16)] = zeros16

        @pl.loop(0, CH_D)
        def _(r):
            ones_v[r, pl.ds(0, 16)] = ones16

        @pl.loop(0, RPT // ZROWS)
        def _(r):
            pltpu.sync_copy(zcnt_v,
                            cnt_sh.at[pl.ds(sid * RPT + r * ZROWS, ZROWS)])

        pltpu.sync_copy(dst_hbm.at[wid], dst_v)

        plsc.subcore_barrier()

        @pl.loop(0, NCH_D)
        def _(j):
            pltpu.sync_copy(ones_v, cnt_sh.at[dst_v.at[j]], add=True)

        plsc.subcore_barrier()

        pltpu.sync_copy(cnt_sh.at[pl.ds(sid * RPT, RPT)],
                        cnt_hbm.at[pl.ds(cid * NP + sid * RPT, RPT)])

    k = pl.kernel(body, out_type=jax.ShapeDtypeStruct((NC * NP, CW),
                                                      jnp.float32),
                  mesh=mesh, scratch_types=scratch,
                  compiler_params=pltpu.CompilerParams(
                      use_tc_tiling_on_sc=False))
    return k(dst3)


_B = 1024


def _sage_block(plo, phi, c0, c1, hlo, hhi, wl_lo, wl_hi, bl, wr_lo, wr_hi):
    cnt = jnp.maximum(c0[0][:, :1] + c1[0][:, :1], 1.0)
    inv = 1.0 / cnt
    out = (jnp.dot(plo[...] * inv, wl_lo[...],
                   preferred_element_type=jnp.float32)
           + jnp.dot(phi[...] * inv, wl_hi[...],
                     preferred_element_type=jnp.float32)
           + jnp.dot(hlo[...], wr_lo[...],
                     preferred_element_type=jnp.float32)
           + jnp.dot(hhi[...], wr_hi[...],
                     preferred_element_type=jnp.float32)
           + bl[...])
    return jnp.maximum(out, 0.0)


def _combine_body(plo, phi, c0, c1, hlo, hhi, wl_lo, wl_hi, bl, wr_lo,
                  wr_hi, olo_ref, ohi_ref):
    res = _sage_block(plo, phi, c0, c1, hlo, hhi, wl_lo, wl_hi, bl,
                      wr_lo, wr_hi)
    olo_ref[...] = res[:, :H]
    ohi_ref[...] = res[:, H:]


def _head_body(plo, phi, c0, c1, hlo, hhi, wl_lo, wl_hi, bl, wr_lo, wr_hi,
               wh1, bh1, wh2, bh2, o_ref):
    res = _sage_block(plo, phi, c0, c1, hlo, hhi, wl_lo, wl_hi, bl,
                      wr_lo, wr_hi)
    t = jnp.maximum(
        jnp.dot(res, wh1[...], preferred_element_type=jnp.float32)
        + bh1[...], 0.0)
    o_ref[...] = (jnp.dot(t, wh2[...], preferred_element_type=jnp.float32)
                  + bh2[...])


def _tc_combine(plo, phi, cnt, hlo, hhi, WlT, bl, WrT, head=None):
    c3 = cnt.reshape(NC, NP, CW)
    bl2 = bl.reshape(1, D)
    full = lambda s: pl.BlockSpec(s, lambda i: (0,) * len(s))
    row = lambda w: pl.BlockSpec((_B, w), lambda i: (i, 0))
    in_specs = [
        row(H), row(H),
        pl.BlockSpec((1, _B, CW), lambda i: (0, i, 0)),
        pl.BlockSpec((1, _B, CW), lambda i: (1, i, 0)),
        row(H), row(H),
        full((H, D)), full((H, D)),
        full((1, D)),
        full((H, D)), full((H, D)),
    ]
    args = [plo, phi, c3, c3, hlo, hhi,
            WlT[:H], WlT[H:], bl2, WrT[:H], WrT[H:]]
    if head is None:
        fn = _combine_body
        out_specs = (row(H), row(H))
        out_shape = (jax.ShapeDtypeStruct((NP, H), jnp.float32),
                     jax.ShapeDtypeStruct((NP, H), jnp.float32))
    else:
        fn = _head_body
        wh1, bh1, wh2, bh2 = head
        in_specs += [full((D, D)), full((1, D)), full((D, D)), full((1, D))]
        args += [wh1, bh1, wh2, bh2]
        out_specs = row(D)
        out_shape = jax.ShapeDtypeStruct((NP, D), jnp.float32)
    return pl.pallas_call(
        fn,
        grid=(NP // _B,),
        in_specs=in_specs,
        out_specs=out_specs,
        out_shape=out_shape,
    )(*args)


def kernel(x, edge_index, Wl0, bl0, Wr0, Wl1, bl1, Wr1, Wl2, bl2, Wr2,
           Wh1, bh1, Wh2, bh2):
    npad = NCH_A * CHUNK - E // NS
    s2 = edge_index[0].reshape(NS, -1)
    d2 = edge_index[1].reshape(NS, -1)
    src16 = jnp.concatenate(
        [s2, jnp.zeros((NS, npad), jnp.int32)], axis=1
    ).reshape(NS, NCH_A, CHUNK)
    dst16 = jnp.concatenate(
        [d2, jnp.full((NS, npad), N, jnp.int32)], axis=1
    ).reshape(NS, NCH_A, CHUNK)
    dst32 = edge_index[1].reshape(NW, NCH_D, CH_D)
    xp = jnp.pad(x, ((0, NP - N), (0, 0)))
    xlo = xp[:, :H]
    xhi = xp[:, H:]

    Wh1T = Wh1.T
    Wh1Tp = jnp.pad(Wh1T, ((0, 0), (0, D - Wh1T.shape[1])))
    bh1p = jnp.pad(bh1, (0, D - bh1.shape[0])).reshape(1, D)
    Wh2T = Wh2.T
    Wh2Tp = jnp.pad(Wh2T, ((0, D - Wh2T.shape[0]), (0, D - Wh2T.shape[1])))
    bh2p = jnp.pad(bh2, (0, D - bh2.shape[0])).reshape(1, D)

    cnt = _sc_degree(dst32)
    plo, phi = _sc_aggregate(xlo, xhi, src16, dst16)
    hlo, hhi = _tc_combine(plo, phi, cnt, xlo, xhi, Wl0.T, bl0, Wr0.T)
    plo, phi = _sc_aggregate(hlo, hhi, src16, dst16)
    hlo, hhi = _tc_combine(plo, phi, cnt, hlo, hhi, Wl1.T, bl1, Wr1.T)
    plo, phi = _sc_aggregate(hlo, hhi, src16, dst16)
    y = _tc_combine(plo, phi, cnt, hlo, hhi, Wl2.T, bl2, Wr2.T,
                    head=(Wh1Tp, bh1p, Wh2Tp, bh2p))
    return y[:N, :4]
